# baseline (speedup 1.0000x reference)
import functools

import jax
import jax.numpy as jnp
from jax import lax
from jax.experimental import pallas as pl
from jax.experimental.pallas import tpu as pltpu
from jax.experimental.pallas import tpu_sc as plsc

N = 10000
E = 320000
D = 128
H = 128
G = 128

NC = 2
NS = 16
NW = NC * NS
CH = 128
CPW = 80
EPW = CH * CPW
EPAD = EPW * NW
NP = N + 8
CORE0_Q = 152
RPT = 632
RPT_LAST = N - 15 * RPT


def _slab_copy(zero_hbm, acc, s, zero_src):
    @pl.when(s < NS - 1)
    def _():
        pltpu.sync_copy(zero_hbm, acc.at[pl.ds(s * RPT, RPT)])

    @pl.when(s == NS - 1)
    def _():
        pltpu.sync_copy(zero_hbm.at[pl.ds(0, RPT_LAST)],
                        acc.at[pl.ds((NS - 1) * RPT, RPT_LAST)])


def _slab_out(acc, out_hbm, c, s):
    @pl.when(s < NS - 1)
    def _():
        pltpu.sync_copy(acc.at[pl.ds(s * RPT, RPT)],
                        out_hbm.at[c, pl.ds(s * RPT, RPT)])

    @pl.when(s == NS - 1)
    def _():
        pltpu.sync_copy(acc.at[pl.ds((NS - 1) * RPT, RPT_LAST)],
                        out_hbm.at[c, pl.ds((NS - 1) * RPT, RPT_LAST)])

@functools.cache
def _sc_kernels():
    mesh = plsc.VectorSubcoreMesh(
        core_axis_name="c", subcore_axis_name="s", num_cores=NC, num_subcores=NS
    )

    CPP = 40
    Q0 = CORE0_Q
    Q1 = 2 * CPW - Q0

    @functools.partial(
        pl.kernel,
        out_type=jax.ShapeDtypeStruct((NC, N, H), jnp.float32),
        mesh=mesh,
        scratch_types=[
            pltpu.VMEM((CPP, CH), jnp.int32),
            pltpu.VMEM((CPP, CH), jnp.int32),
            pltpu.VMEM((CH, H), jnp.float32),
            pltpu.VMEM((CH, H), jnp.float32),
            pltpu.VMEM_SHARED((NP, H), jnp.float32),
        ]
        + [pltpu.SemaphoreType.DMA] * 4,
    )
    def _sc_agg(h_hbm, src_hbm, dst_hbm, zero_hbm, out_hbm,
                sidx, didx, r0, r1, acc, g0, g1, s0, s1):
        c = lax.axis_index("c")
        s = lax.axis_index("s")
        rows = [r0, r1]
        gsem = [g0, g1]
        ssem = [s0, s1]

        _slab_copy(zero_hbm, acc, s, zero_src=True)
        plsc.subcore_barrier()

        def run_phase(base, L):
            pltpu.sync_copy(src_hbm.at[pl.ds(base, L)], sidx.at[pl.ds(0, L)])
            pltpu.sync_copy(dst_hbm.at[pl.ds(base, L)], didx.at[pl.ds(0, L)])
            pltpu.async_copy(h_hbm.at[sidx.at[0]], rows[0], gsem[0])

            @pl.loop(0, L // 2)
            def _chunks(tj):
                for b in range(2):
                    t = tj * 2 + b
                    pltpu.make_async_copy(h_hbm.at[sidx.at[0]], rows[b],
                                          gsem[b]).wait()
                    pltpu.async_copy(rows[b], acc.at[didx.at[t]], ssem[b],
                                     add=True)

                    @pl.when(t + 1 < L)
                    def _():
                        @pl.when(t >= 1)
                        def _():
                            pltpu.make_async_copy(rows[1 - b],
                                                  acc.at[didx.at[0]],
                                                  ssem[1 - b]).wait()

                        pltpu.async_copy(h_hbm.at[sidx.at[t + 1]],
                                         rows[1 - b], gsem[1 - b])

            bl = (L - 1) % 2
            pltpu.make_async_copy(rows[bl], acc.at[didx.at[0]], ssem[bl]).wait()

        def run_chunks(base0, q):
            done = 0
            while done < q:
                L = min(CPP, q - done)
                run_phase(base0 + done, L)
                done += L

        @pl.when(c == 0)
        def _():
            run_chunks(s * Q0, Q0)

        @pl.when(c == 1)
        def _():
            run_chunks(NS * Q0 + s * Q1, Q1)

        plsc.subcore_barrier()
        _slab_out(acc, out_hbm, c, s)

    return _sc_agg


BR = 2000

NHI = 80
BE = 8000


def _tc_deg_body(dst_ref, o_ref, acc):
    i = pl.program_id(0)

    @pl.when(i == 0)
    def _():
        acc[...] = jnp.zeros_like(acc)

    d = dst_ref[...]
    hi = d // 128
    lo = d - hi * 128
    oh_hi = (hi == lax.broadcasted_iota(jnp.int32, (BE, NHI), 1)).astype(jnp.float32)
    oh_lo = (lo == lax.broadcasted_iota(jnp.int32, (BE, 128), 1)).astype(jnp.float32)
    dn = (((0,), (0,)), ((), ()))
    acc[...] += lax.dot_general(oh_hi, oh_lo, dn, preferred_element_type=jnp.float32)

    @pl.when(i == pl.num_programs(0) - 1)
    def _():
        o_ref[...] = acc[...]


def _tc_deg(dst2d):
    return pl.pallas_call(
        _tc_deg_body,
        grid=(E // BE,),
        in_specs=[pl.BlockSpec((BE, 1), lambda i: (i, 0))],
        out_specs=pl.BlockSpec((NHI, 128), lambda i: (0, 0)),
        out_shape=jax.ShapeDtypeStruct((NHI, 128), jnp.float32),
        scratch_shapes=[pltpu.VMEM((NHI, 128), jnp.float32)],
    )(dst2d)


def _tc_in_body(x_ref, w1, b1, s1, t1, w2, b2, o_ref):
    z = jnp.dot(x_ref[...], w1[...], preferred_element_type=jnp.float32) + b1[...]
    z = jnp.maximum(z * s1[...] + t1[...], 0.0)
    o_ref[...] = jnp.dot(z, w2[...], preferred_element_type=jnp.float32) + b2[...]


def _tc_in(x, w1, b1, s1, t1, w2, b2):
    full = lambda r, c: pl.BlockSpec((r, c), lambda i: (0, 0))
    return pl.pallas_call(
        _tc_in_body,
        grid=(N // BR,),
        in_specs=[
            pl.BlockSpec((BR, D), lambda i: (i, 0)),
            full(D, H), full(1, H), full(1, H), full(1, H), full(H, H), full(1, H),
        ],
        out_specs=pl.BlockSpec((BR, H), lambda i: (i, 0)),
        out_shape=jax.ShapeDtypeStruct((N, H), jnp.float32),
    )(x, w1, b1, s1, t1, w2, b2)


def _tc_gin_body(h_ref, p0, p1, d_ref, w1, b1, w2, b2, sb, tb, o_ref):
    inv = 1.0 / jnp.maximum(d_ref[...], 1.0)
    z = h_ref[...] + (p0[...] + p1[...]) * inv
    z = jnp.maximum(
        jnp.dot(z, w1[...], preferred_element_type=jnp.float32) + b1[...], 0.0)
    z = jnp.dot(z, w2[...], preferred_element_type=jnp.float32) + b2[...]
    o_ref[...] = jnp.maximum(z * sb[...] + tb[...], 0.0)


def _tc_gin(h, p0, p1, deg, w1, b1, w2, b2, sb, tb):
    full = lambda r, c: pl.BlockSpec((r, c), lambda i: (0, 0))
    row = lambda c: pl.BlockSpec((BR, c), lambda i: (i, 0))
    return pl.pallas_call(
        _tc_gin_body,
        grid=(N // BR,),
        in_specs=[
            row(H), row(H), row(H), row(1),
            full(H, H), full(1, H), full(H, H), full(1, H), full(1, H), full(1, H),
        ],
        out_specs=row(H),
        out_shape=jax.ShapeDtypeStruct((N, H), jnp.float32),
    )(h, p0, p1, deg, w1, b1, w2, b2, sb, tb)


def _tc_pool_body(h0, h1, h2, h3, b_ref,
                  pw0, pb0, pw1, pb1, pw2, pb2, pw3, pb3,
                  np_ref, gp_ref, a0, a1, a2, a3, cnt):
    i = pl.program_id(0)

    @pl.when(i == 0)
    def _():
        a0[...] = jnp.zeros_like(a0)
        a1[...] = jnp.zeros_like(a1)
        a2[...] = jnp.zeros_like(a2)
        a3[...] = jnp.zeros_like(a3)
        cnt[...] = jnp.zeros_like(cnt)

    oh = (b_ref[...] == lax.broadcasted_iota(jnp.int32, (BR, G), 1)
          ).astype(jnp.float32)
    dn = (((0,), (0,)), ((), ()))
    a0[...] += lax.dot_general(oh, h0[...], dn, preferred_element_type=jnp.float32)
    a1[...] += lax.dot_general(oh, h1[...], dn, preferred_element_type=jnp.float32)
    a2[...] += lax.dot_general(oh, h2[...], dn, preferred_element_type=jnp.float32)
    a3[...] += lax.dot_general(oh, h3[...], dn, preferred_element_type=jnp.float32)
    cnt[...] += lax.dot_general(oh, jnp.ones((BR, 1), jnp.float32), dn,
                                preferred_element_type=jnp.float32)
    np_ref[...] = h1[...] + h2[...] + h3[...]

    @pl.when(i == pl.num_programs(0) - 1)
    def _():
        invc = 1.0 / jnp.maximum(cnt[...], 1.0)
        g = jnp.dot(a0[...] * invc, pw0[...], preferred_element_type=jnp.float32) + pb0[...]
        g += jnp.dot(a1[...] * invc, pw1[...], preferred_element_type=jnp.float32) + pb1[...]
        g += jnp.dot(a2[...] * invc, pw2[...], preferred_element_type=jnp.float32) + pb2[...]
        g += jnp.dot(a3[...] * invc, pw3[...], preferred_element_type=jnp.float32) + pb3[...]
        gp_ref[...] = g


def _tc_pool(h0, h1, h2, h3, batch2d, preds):
    full = lambda r, c: pl.BlockSpec((r, c), lambda i: (0, 0))
    row = lambda c: pl.BlockSpec((BR, c), lambda i: (i, 0))
    return pl.pallas_call(
        _tc_pool_body,
        grid=(N // BR,),
        in_specs=[row(H), row(H), row(H), row(H), row(1)]
        + [full(H, H) if k % 2 == 0 else full(1, H) for k in range(8)],
        out_specs=[row(H), full(G, H)],
        out_shape=[
            jax.ShapeDtypeStruct((N, H), jnp.float32),
            jax.ShapeDtypeStruct((G, H), jnp.float32),
        ],
        scratch_shapes=[pltpu.VMEM((G, H), jnp.float32)] * 4
        + [pltpu.VMEM((G, 1), jnp.float32)],
    )(h0, h1, h2, h3, batch2d, *preds)


_BN_S = 1.0 / (1.0 + 1e-5) ** 0.5


def kernel(x, params, edge_index, batch):
    p = params
    pad = EPAD - E
    src = edge_index[0].astype(jnp.int32)
    dst = edge_index[1].astype(jnp.int32)
    srcp = jnp.concatenate([src, jnp.zeros((pad,), jnp.int32)]).reshape(EPAD // CH, CH)
    dstp = jnp.concatenate([dst, jnp.full((pad,), N, jnp.int32)]).reshape(EPAD // CH, CH)
    batch2d = batch.astype(jnp.int32).reshape(N, 1)

    zero_h = jnp.zeros((RPT, H), jnp.float32)

    r = lambda v: v.reshape(1, H)
    sb1 = r(p['bn1_1_g'] * _BN_S)
    tb1 = r(p['bn1_1_b'])

    sc_agg = _sc_kernels()
    deg = _tc_deg(dst.reshape(E, 1)).reshape(NHI * 128, 1)[:N]

    h0 = _tc_in(x, p['lin1_1_W'], r(p['lin1_1_b']), sb1, tb1,
                p['lin1_2_W'], r(p['lin1_2_b']))

    hs = [h0]
    h = h0
    for i in range(1, 4):
        ag = sc_agg(h, srcp, dstp, zero_h)
        h = _tc_gin(h, ag[0], ag[1], deg,
                    p[f'conv{i}_W1'], r(p[f'conv{i}_b1']),
                    p[f'conv{i}_W2'], r(p[f'conv{i}_b2']),
                    r(p[f'bn{i}_g'] * _BN_S), r(p[f'bn{i}_b']))
        hs.append(h)

    preds = []
    for l in range(4):
        preds += [p[f'pred{l}_W'], r(p[f'pred{l}_b'])]
    npool, gpool = _tc_pool(hs[0], hs[1], hs[2], hs[3], batch2d, preds)
    return (npool, gpool)


# speedup vs baseline: 1.0121x; 1.0121x over previous
import functools

import jax
import jax.numpy as jnp
from jax import lax
from jax.experimental import pallas as pl
from jax.experimental.pallas import tpu as pltpu
from jax.experimental.pallas import tpu_sc as plsc

N = 10000
E = 320000
D = 128
H = 128
G = 128

NC = 2
NS = 16
NW = NC * NS
CH = 128
CPW = 80
EPW = CH * CPW
EPAD = EPW * NW
NP = N + 8
CORE0_Q = 144
RPT = 632
RPT_LAST = N - 15 * RPT


def _slab_copy(zero_hbm, acc, s, zero_src):
    @pl.when(s < NS - 1)
    def _():
        pltpu.sync_copy(zero_hbm, acc.at[pl.ds(s * RPT, RPT)])

    @pl.when(s == NS - 1)
    def _():
        pltpu.sync_copy(zero_hbm.at[pl.ds(0, RPT_LAST)],
                        acc.at[pl.ds((NS - 1) * RPT, RPT_LAST)])


def _slab_out(acc, out_hbm, c, s):
    @pl.when(s < NS - 1)
    def _():
        pltpu.sync_copy(acc.at[pl.ds(s * RPT, RPT)],
                        out_hbm.at[c, pl.ds(s * RPT, RPT)])

    @pl.when(s == NS - 1)
    def _():
        pltpu.sync_copy(acc.at[pl.ds((NS - 1) * RPT, RPT_LAST)],
                        out_hbm.at[c, pl.ds((NS - 1) * RPT, RPT_LAST)])

@functools.cache
def _sc_kernels():
    mesh = plsc.VectorSubcoreMesh(
        core_axis_name="c", subcore_axis_name="s", num_cores=NC, num_subcores=NS
    )

    CPP = 48
    Q0 = CORE0_Q
    Q1 = 2 * CPW - Q0

    @functools.partial(
        pl.kernel,
        out_type=jax.ShapeDtypeStruct((NC, N, H), jnp.float32),
        mesh=mesh,
        scratch_types=[
            pltpu.VMEM((CPP, CH), jnp.int32),
            pltpu.VMEM((CPP, CH), jnp.int32),
            pltpu.VMEM((CH, H), jnp.float32),
            pltpu.VMEM((CH, H), jnp.float32),
            pltpu.VMEM_SHARED((NP, H), jnp.float32),
        ]
        + [pltpu.SemaphoreType.DMA] * 4,
    )
    def _sc_agg(h_hbm, src_hbm, dst_hbm, zero_hbm, out_hbm,
                sidx, didx, r0, r1, acc, g0, g1, s0, s1):
        c = lax.axis_index("c")
        s = lax.axis_index("s")
        rows = [r0, r1]
        gsem = [g0, g1]
        ssem = [s0, s1]

        _slab_copy(zero_hbm, acc, s, zero_src=True)
        plsc.subcore_barrier()

        def run_phase(base, L):
            pltpu.sync_copy(src_hbm.at[pl.ds(base, L)], sidx.at[pl.ds(0, L)])
            pltpu.sync_copy(dst_hbm.at[pl.ds(base, L)], didx.at[pl.ds(0, L)])
            pltpu.async_copy(h_hbm.at[sidx.at[0]], rows[0], gsem[0])

            @pl.loop(0, L // 2)
            def _chunks(tj):
                for b in range(2):
                    t = tj * 2 + b
                    pltpu.make_async_copy(h_hbm.at[sidx.at[0]], rows[b],
                                          gsem[b]).wait()
                    pltpu.async_copy(rows[b], acc.at[didx.at[t]], ssem[b],
                                     add=True)

                    @pl.when(t + 1 < L)
                    def _():
                        @pl.when(t >= 1)
                        def _():
                            pltpu.make_async_copy(rows[1 - b],
                                                  acc.at[didx.at[0]],
                                                  ssem[1 - b]).wait()

                        pltpu.async_copy(h_hbm.at[sidx.at[t + 1]],
                                         rows[1 - b], gsem[1 - b])

            bl = (L - 1) % 2
            pltpu.make_async_copy(rows[bl], acc.at[didx.at[0]], ssem[bl]).wait()

        def run_chunks(base0, q):
            done = 0
            while done < q:
                L = min(CPP, q - done)
                run_phase(base0 + done, L)
                done += L

        @pl.when(c == 0)
        def _():
            run_chunks(s * Q0, Q0)

        @pl.when(c == 1)
        def _():
            run_chunks(NS * Q0 + s * Q1, Q1)

        plsc.subcore_barrier()
        _slab_out(acc, out_hbm, c, s)

    return _sc_agg


BR = 2000

NHI = 80
BE = 8000


def _tc_deg_body(dst_ref, o_ref, acc):
    i = pl.program_id(0)

    @pl.when(i == 0)
    def _():
        acc[...] = jnp.zeros_like(acc)

    d = dst_ref[...]
    hi = d // 128
    lo = d - hi * 128
    oh_hi = (hi == lax.broadcasted_iota(jnp.int32, (BE, NHI), 1)).astype(jnp.float32)
    oh_lo = (lo == lax.broadcasted_iota(jnp.int32, (BE, 128), 1)).astype(jnp.float32)
    dn = (((0,), (0,)), ((), ()))
    acc[...] += lax.dot_general(oh_hi, oh_lo, dn, preferred_element_type=jnp.float32)

    @pl.when(i == pl.num_programs(0) - 1)
    def _():
        o_ref[...] = acc[...]


def _tc_deg(dst2d):
    return pl.pallas_call(
        _tc_deg_body,
        grid=(E // BE,),
        in_specs=[pl.BlockSpec((BE, 1), lambda i: (i, 0))],
        out_specs=pl.BlockSpec((NHI, 128), lambda i: (0, 0)),
        out_shape=jax.ShapeDtypeStruct((NHI, 128), jnp.float32),
        scratch_shapes=[pltpu.VMEM((NHI, 128), jnp.float32)],
    )(dst2d)


def _tc_in_body(x_ref, w1, b1, s1, t1, w2, b2, o_ref):
    z = jnp.dot(x_ref[...], w1[...], preferred_element_type=jnp.float32) + b1[...]
    z = jnp.maximum(z * s1[...] + t1[...], 0.0)
    o_ref[...] = jnp.dot(z, w2[...], preferred_element_type=jnp.float32) + b2[...]


def _tc_in(x, w1, b1, s1, t1, w2, b2):
    full = lambda r, c: pl.BlockSpec((r, c), lambda i: (0, 0))
    return pl.pallas_call(
        _tc_in_body,
        grid=(N // BR,),
        in_specs=[
            pl.BlockSpec((BR, D), lambda i: (i, 0)),
            full(D, H), full(1, H), full(1, H), full(1, H), full(H, H), full(1, H),
        ],
        out_specs=pl.BlockSpec((BR, H), lambda i: (i, 0)),
        out_shape=jax.ShapeDtypeStruct((N, H), jnp.float32),
    )(x, w1, b1, s1, t1, w2, b2)


def _tc_gin_body(h_ref, p0, p1, d_ref, w1, b1, w2, b2, sb, tb, o_ref):
    inv = 1.0 / jnp.maximum(d_ref[...], 1.0)
    z = h_ref[...] + (p0[...] + p1[...]) * inv
    z = jnp.maximum(
        jnp.dot(z, w1[...], preferred_element_type=jnp.float32) + b1[...], 0.0)
    z = jnp.dot(z, w2[...], preferred_element_type=jnp.float32) + b2[...]
    o_ref[...] = jnp.maximum(z * sb[...] + tb[...], 0.0)


def _tc_gin(h, p0, p1, deg, w1, b1, w2, b2, sb, tb):
    full = lambda r, c: pl.BlockSpec((r, c), lambda i: (0, 0))
    row = lambda c: pl.BlockSpec((BR, c), lambda i: (i, 0))
    return pl.pallas_call(
        _tc_gin_body,
        grid=(N // BR,),
        in_specs=[
            row(H), row(H), row(H), row(1),
            full(H, H), full(1, H), full(H, H), full(1, H), full(1, H), full(1, H),
        ],
        out_specs=row(H),
        out_shape=jax.ShapeDtypeStruct((N, H), jnp.float32),
    )(h, p0, p1, deg, w1, b1, w2, b2, sb, tb)


def _tc_pool_body(h0, h1, h2, h3, b_ref,
                  pw0, pb0, pw1, pb1, pw2, pb2, pw3, pb3,
                  np_ref, gp_ref, a0, a1, a2, a3, cnt):
    i = pl.program_id(0)

    @pl.when(i == 0)
    def _():
        a0[...] = jnp.zeros_like(a0)
        a1[...] = jnp.zeros_like(a1)
        a2[...] = jnp.zeros_like(a2)
        a3[...] = jnp.zeros_like(a3)
        cnt[...] = jnp.zeros_like(cnt)

    oh = (b_ref[...] == lax.broadcasted_iota(jnp.int32, (BR, G), 1)
          ).astype(jnp.float32)
    dn = (((0,), (0,)), ((), ()))
    a0[...] += lax.dot_general(oh, h0[...], dn, preferred_element_type=jnp.float32)
    a1[...] += lax.dot_general(oh, h1[...], dn, preferred_element_type=jnp.float32)
    a2[...] += lax.dot_general(oh, h2[...], dn, preferred_element_type=jnp.float32)
    a3[...] += lax.dot_general(oh, h3[...], dn, preferred_element_type=jnp.float32)
    cnt[...] += lax.dot_general(oh, jnp.ones((BR, 1), jnp.float32), dn,
                                preferred_element_type=jnp.float32)
    np_ref[...] = h1[...] + h2[...] + h3[...]

    @pl.when(i == pl.num_programs(0) - 1)
    def _():
        invc = 1.0 / jnp.maximum(cnt[...], 1.0)
        g = jnp.dot(a0[...] * invc, pw0[...], preferred_element_type=jnp.float32) + pb0[...]
        g += jnp.dot(a1[...] * invc, pw1[...], preferred_element_type=jnp.float32) + pb1[...]
        g += jnp.dot(a2[...] * invc, pw2[...], preferred_element_type=jnp.float32) + pb2[...]
        g += jnp.dot(a3[...] * invc, pw3[...], preferred_element_type=jnp.float32) + pb3[...]
        gp_ref[...] = g


def _tc_pool(h0, h1, h2, h3, batch2d, preds):
    full = lambda r, c: pl.BlockSpec((r, c), lambda i: (0, 0))
    row = lambda c: pl.BlockSpec((BR, c), lambda i: (i, 0))
    return pl.pallas_call(
        _tc_pool_body,
        grid=(N // BR,),
        in_specs=[row(H), row(H), row(H), row(H), row(1)]
        + [full(H, H) if k % 2 == 0 else full(1, H) for k in range(8)],
        out_specs=[row(H), full(G, H)],
        out_shape=[
            jax.ShapeDtypeStruct((N, H), jnp.float32),
            jax.ShapeDtypeStruct((G, H), jnp.float32),
        ],
        scratch_shapes=[pltpu.VMEM((G, H), jnp.float32)] * 4
        + [pltpu.VMEM((G, 1), jnp.float32)],
    )(h0, h1, h2, h3, batch2d, *preds)


_BN_S = 1.0 / (1.0 + 1e-5) ** 0.5


def kernel(x, params, edge_index, batch):
    p = params
    pad = EPAD - E
    src = edge_index[0].astype(jnp.int32)
    dst = edge_index[1].astype(jnp.int32)
    srcp = jnp.concatenate([src, jnp.zeros((pad,), jnp.int32)]).reshape(EPAD // CH, CH)
    dstp = jnp.concatenate([dst, jnp.full((pad,), N, jnp.int32)]).reshape(EPAD // CH, CH)
    batch2d = batch.astype(jnp.int32).reshape(N, 1)

    zero_h = jnp.zeros((RPT, H), jnp.float32)

    r = lambda v: v.reshape(1, H)
    sb1 = r(p['bn1_1_g'] * _BN_S)
    tb1 = r(p['bn1_1_b'])

    sc_agg = _sc_kernels()
    deg = _tc_deg(dst.reshape(E, 1)).reshape(NHI * 128, 1)[:N]

    h0 = _tc_in(x, p['lin1_1_W'], r(p['lin1_1_b']), sb1, tb1,
                p['lin1_2_W'], r(p['lin1_2_b']))

    hs = [h0]
    h = h0
    for i in range(1, 4):
        ag = sc_agg(h, srcp, dstp, zero_h)
        h = _tc_gin(h, ag[0], ag[1], deg,
                    p[f'conv{i}_W1'], r(p[f'conv{i}_b1']),
                    p[f'conv{i}_W2'], r(p[f'conv{i}_b2']),
                    r(p[f'bn{i}_g'] * _BN_S), r(p[f'bn{i}_b']))
        hs.append(h)

    preds = []
    for l in range(4):
        preds += [p[f'pred{l}_W'], r(p[f'pred{l}_b'])]
    npool, gpool = _tc_pool(hs[0], hs[1], hs[2], hs[3], batch2d, preds)
    return (npool, gpool)


# speedup vs baseline: 1.0183x; 1.0061x over previous
import functools

import jax
import jax.numpy as jnp
from jax import lax
from jax.experimental import pallas as pl
from jax.experimental.pallas import tpu as pltpu
from jax.experimental.pallas import tpu_sc as plsc

N = 10000
E = 320000
D = 128
H = 128
G = 128

NC = 2
NS = 16
NW = NC * NS
CH = 128
CPW = 80
EPW = CH * CPW
EPAD = EPW * NW
NP = N + 8
CORE0_Q = 144
RPT = 632
RPT_LAST = N - 15 * RPT


def _slab_copy(zero_hbm, acc, s, zero_src):
    @pl.when(s < NS - 1)
    def _():
        pltpu.sync_copy(zero_hbm, acc.at[pl.ds(s * RPT, RPT)])

    @pl.when(s == NS - 1)
    def _():
        pltpu.sync_copy(zero_hbm.at[pl.ds(0, RPT_LAST)],
                        acc.at[pl.ds((NS - 1) * RPT, RPT_LAST)])


def _slab_out(acc, out_hbm, c, s):
    @pl.when(s < NS - 1)
    def _():
        pltpu.sync_copy(acc.at[pl.ds(s * RPT, RPT)],
                        out_hbm.at[c, pl.ds(s * RPT, RPT)])

    @pl.when(s == NS - 1)
    def _():
        pltpu.sync_copy(acc.at[pl.ds((NS - 1) * RPT, RPT_LAST)],
                        out_hbm.at[c, pl.ds((NS - 1) * RPT, RPT_LAST)])

@functools.cache
def _sc_kernels():
    mesh = plsc.VectorSubcoreMesh(
        core_axis_name="c", subcore_axis_name="s", num_cores=NC, num_subcores=NS
    )

    CPP = 48
    Q0 = CORE0_Q
    Q1 = 2 * CPW - Q0

    @functools.partial(
        pl.kernel,
        out_type=jax.ShapeDtypeStruct((NC, N, H), jnp.float32),
        mesh=mesh,
        scratch_types=[
            pltpu.VMEM((CPP, CH), jnp.int32),
            pltpu.VMEM((CPP, CH), jnp.int32),
            pltpu.VMEM((CH, H), jnp.float32),
            pltpu.VMEM((CH, H), jnp.float32),
            pltpu.VMEM_SHARED((NP, H), jnp.float32),
        ]
        + [pltpu.SemaphoreType.DMA] * 4,
    )
    def _sc_agg(h_hbm, src_hbm, dst_hbm, zero_hbm, out_hbm,
                sidx, didx, r0, r1, acc, g0, g1, s0, s1):
        c = lax.axis_index("c")
        s = lax.axis_index("s")
        rows = [r0, r1]
        gsem = [g0, g1]
        ssem = [s0, s1]

        _slab_copy(zero_hbm, acc, s, zero_src=True)
        plsc.subcore_barrier()

        def run_phase(base, L):
            pltpu.sync_copy(src_hbm.at[pl.ds(base, L)], sidx.at[pl.ds(0, L)])
            pltpu.sync_copy(dst_hbm.at[pl.ds(base, L)], didx.at[pl.ds(0, L)])
            pltpu.async_copy(h_hbm.at[sidx.at[0]], rows[0], gsem[0])

            @pl.loop(0, L // 2)
            def _chunks(tj):
                for b in range(2):
                    t = tj * 2 + b
                    pltpu.make_async_copy(h_hbm.at[sidx.at[0]], rows[b],
                                          gsem[b]).wait()
                    pltpu.async_copy(rows[b], acc.at[didx.at[t]], ssem[b],
                                     add=True)

                    @pl.when(t + 1 < L)
                    def _():
                        @pl.when(t >= 1)
                        def _():
                            pltpu.make_async_copy(rows[1 - b],
                                                  acc.at[didx.at[0]],
                                                  ssem[1 - b]).wait()

                        pltpu.async_copy(h_hbm.at[sidx.at[t + 1]],
                                         rows[1 - b], gsem[1 - b])

            bl = (L - 1) % 2
            pltpu.make_async_copy(rows[bl], acc.at[didx.at[0]], ssem[bl]).wait()

        def run_chunks(base0, q):
            done = 0
            while done < q:
                L = min(CPP, q - done)
                run_phase(base0 + done, L)
                done += L

        @pl.when(c == 0)
        def _():
            run_chunks(s * Q0, Q0)

        @pl.when(c == 1)
        def _():
            run_chunks(NS * Q0 + s * Q1, Q1)

        plsc.subcore_barrier()
        _slab_out(acc, out_hbm, c, s)

    return _sc_agg


BR = 2000

NHI = 80
BE = 8000


def _tc_deg_body(dst_ref, o_ref, acc):
    i = pl.program_id(0)

    @pl.when(i == 0)
    def _():
        acc[...] = jnp.zeros_like(acc)

    d = dst_ref[...]
    hi = d // 128
    lo = d - hi * 128
    oh_hi = (hi == lax.broadcasted_iota(jnp.int32, (BE, NHI), 1)).astype(jnp.float32)
    oh_lo = (lo == lax.broadcasted_iota(jnp.int32, (BE, 128), 1)).astype(jnp.float32)
    dn = (((0,), (0,)), ((), ()))
    acc[...] += lax.dot_general(oh_hi, oh_lo, dn, preferred_element_type=jnp.float32)

    @pl.when(i == pl.num_programs(0) - 1)
    def _():
        o_ref[...] = acc[...]


def _tc_deg(dst2d):
    return pl.pallas_call(
        _tc_deg_body,
        grid=(E // BE,),
        in_specs=[pl.BlockSpec((BE, 1), lambda i: (i, 0))],
        out_specs=pl.BlockSpec((NHI, 128), lambda i: (0, 0)),
        out_shape=jax.ShapeDtypeStruct((NHI, 128), jnp.float32),
        scratch_shapes=[pltpu.VMEM((NHI, 128), jnp.float32)],
    )(dst2d)


def _tc_in_body(x_ref, w1, b1, s1, t1, w2, b2, o_ref):
    z = jnp.dot(x_ref[...], w1[...], preferred_element_type=jnp.float32) + b1[...]
    z = jnp.maximum(z * s1[...] + t1[...], 0.0)
    o_ref[...] = jnp.dot(z, w2[...], preferred_element_type=jnp.float32) + b2[...]


def _tc_in(x, w1, b1, s1, t1, w2, b2):
    full = lambda r, c: pl.BlockSpec((r, c), lambda i: (0, 0))
    return pl.pallas_call(
        _tc_in_body,
        grid=(N // BR,),
        in_specs=[
            pl.BlockSpec((BR, D), lambda i: (i, 0)),
            full(D, H), full(1, H), full(1, H), full(1, H), full(H, H), full(1, H),
        ],
        out_specs=pl.BlockSpec((BR, H), lambda i: (i, 0)),
        out_shape=jax.ShapeDtypeStruct((N, H), jnp.float32),
    )(x, w1, b1, s1, t1, w2, b2)


def _tc_gin_body(h_ref, p0, p1, d_ref, w1, b1, w2, b2, sb, tb, o_ref):
    inv = 1.0 / jnp.maximum(d_ref[...], 1.0)
    z = h_ref[...] + (p0[...] + p1[...]) * inv
    z = jnp.maximum(
        jnp.dot(z, w1[...], preferred_element_type=jnp.float32) + b1[...], 0.0)
    z = jnp.dot(z, w2[...], preferred_element_type=jnp.float32) + b2[...]
    o_ref[...] = jnp.maximum(z * sb[...] + tb[...], 0.0)


def _tc_gin(h, p0, p1, deg, w1, b1, w2, b2, sb, tb):
    full = lambda r, c: pl.BlockSpec((r, c), lambda i: (0, 0))
    row = lambda c: pl.BlockSpec((BR, c), lambda i: (i, 0))
    return pl.pallas_call(
        _tc_gin_body,
        grid=(N // BR,),
        in_specs=[
            row(H), row(H), row(H), row(1),
            full(H, H), full(1, H), full(H, H), full(1, H), full(1, H), full(1, H),
        ],
        out_specs=row(H),
        out_shape=jax.ShapeDtypeStruct((N, H), jnp.float32),
    )(h, p0, p1, deg, w1, b1, w2, b2, sb, tb)


def _tc_pool_body(h0, h1, h2, p0, p1, d_ref, w1, b1, w2, b2, sb, tb, b_ref,
                  pw0, pb0, pw1, pb1, pw2, pb2, pw3, pb3,
                  np_ref, gp_ref, a0, a1, a2, a3, cnt):
    i = pl.program_id(0)

    inv = 1.0 / jnp.maximum(d_ref[...], 1.0)
    z = h2[...] + (p0[...] + p1[...]) * inv
    z = jnp.maximum(
        jnp.dot(z, w1[...], preferred_element_type=jnp.float32) + b1[...], 0.0)
    z = jnp.dot(z, w2[...], preferred_element_type=jnp.float32) + b2[...]
    h3 = jnp.maximum(z * sb[...] + tb[...], 0.0)

    @pl.when(i == 0)
    def _():
        a0[...] = jnp.zeros_like(a0)
        a1[...] = jnp.zeros_like(a1)
        a2[...] = jnp.zeros_like(a2)
        a3[...] = jnp.zeros_like(a3)
        cnt[...] = jnp.zeros_like(cnt)

    oh = (b_ref[...] == lax.broadcasted_iota(jnp.int32, (BR, G), 1)
          ).astype(jnp.float32)
    dn = (((0,), (0,)), ((), ()))
    a0[...] += lax.dot_general(oh, h0[...], dn, preferred_element_type=jnp.float32)
    a1[...] += lax.dot_general(oh, h1[...], dn, preferred_element_type=jnp.float32)
    a2[...] += lax.dot_general(oh, h2[...], dn, preferred_element_type=jnp.float32)
    a3[...] += lax.dot_general(oh, h3, dn, preferred_element_type=jnp.float32)
    cnt[...] += lax.dot_general(oh, jnp.ones((BR, 1), jnp.float32), dn,
                                preferred_element_type=jnp.float32)
    np_ref[...] = h1[...] + h2[...] + h3

    @pl.when(i == pl.num_programs(0) - 1)
    def _():
        invc = 1.0 / jnp.maximum(cnt[...], 1.0)
        g = jnp.dot(a0[...] * invc, pw0[...], preferred_element_type=jnp.float32) + pb0[...]
        g += jnp.dot(a1[...] * invc, pw1[...], preferred_element_type=jnp.float32) + pb1[...]
        g += jnp.dot(a2[...] * invc, pw2[...], preferred_element_type=jnp.float32) + pb2[...]
        g += jnp.dot(a3[...] * invc, pw3[...], preferred_element_type=jnp.float32) + pb3[...]
        gp_ref[...] = g


def _tc_pool(h0, h1, h2, p0, p1, deg, conv3, batch2d, preds):
    full = lambda r, c: pl.BlockSpec((r, c), lambda i: (0, 0))
    row = lambda c: pl.BlockSpec((BR, c), lambda i: (i, 0))
    return pl.pallas_call(
        _tc_pool_body,
        grid=(N // BR,),
        in_specs=[row(H), row(H), row(H), row(H), row(H), row(1)]
        + [full(H, H), full(1, H), full(H, H), full(1, H), full(1, H), full(1, H)]
        + [row(1)]
        + [full(H, H) if k % 2 == 0 else full(1, H) for k in range(8)],
        out_specs=[row(H), full(G, H)],
        out_shape=[
            jax.ShapeDtypeStruct((N, H), jnp.float32),
            jax.ShapeDtypeStruct((G, H), jnp.float32),
        ],
        scratch_shapes=[pltpu.VMEM((G, H), jnp.float32)] * 4
        + [pltpu.VMEM((G, 1), jnp.float32)],
    )(h0, h1, h2, p0, p1, deg, *conv3, batch2d, *preds)


_BN_S = 1.0 / (1.0 + 1e-5) ** 0.5


def kernel(x, params, edge_index, batch):
    p = params
    pad = EPAD - E
    src = edge_index[0].astype(jnp.int32)
    dst = edge_index[1].astype(jnp.int32)
    srcp = jnp.concatenate([src, jnp.zeros((pad,), jnp.int32)]).reshape(EPAD // CH, CH)
    dstp = jnp.concatenate([dst, jnp.full((pad,), N, jnp.int32)]).reshape(EPAD // CH, CH)
    batch2d = batch.astype(jnp.int32).reshape(N, 1)

    zero_h = jnp.zeros((RPT, H), jnp.float32)

    r = lambda v: v.reshape(1, H)
    sb1 = r(p['bn1_1_g'] * _BN_S)
    tb1 = r(p['bn1_1_b'])

    sc_agg = _sc_kernels()
    deg = _tc_deg(dst.reshape(E, 1)).reshape(NHI * 128, 1)[:N]

    h0 = _tc_in(x, p['lin1_1_W'], r(p['lin1_1_b']), sb1, tb1,
                p['lin1_2_W'], r(p['lin1_2_b']))

    hs = [h0]
    h = h0
    for i in range(1, 3):
        ag = sc_agg(h, srcp, dstp, zero_h)
        h = _tc_gin(h, ag[0], ag[1], deg,
                    p[f'conv{i}_W1'], r(p[f'conv{i}_b1']),
                    p[f'conv{i}_W2'], r(p[f'conv{i}_b2']),
                    r(p[f'bn{i}_g'] * _BN_S), r(p[f'bn{i}_b']))
        hs.append(h)

    ag3 = sc_agg(h, srcp, dstp, zero_h)
    conv3 = (p['conv3_W1'], r(p['conv3_b1']), p['conv3_W2'], r(p['conv3_b2']),
             r(p['bn3_g'] * _BN_S), r(p['bn3_b']))
    preds = []
    for l in range(4):
        preds += [p[f'pred{l}_W'], r(p[f'pred{l}_b'])]
    npool, gpool = _tc_pool(hs[0], hs[1], hs[2], ag3[0], ag3[1], deg,
                            conv3, batch2d, preds)
    return (npool, gpool)
